# baseline (speedup 1.0000x reference)
import jax
import jax.numpy as jnp
from jax import lax
from jax.experimental import pallas as pl
from jax.experimental.pallas import tpu as pltpu
from jax.experimental.pallas import tpu_sc as plsc

B = 128
N = 32768
K = 256
NW = 32
R_PER = B // NW
NB = 1024
SEG = 768
CAP = 2 * SEG
NVREG = CAP // 16
NEG_INF = float("-inf")
BIG = 2 ** 30


def _key16(v):
    xi = lax.bitcast_convert_type(v, jnp.int32)
    s = lax.shift_right_arithmetic(xi, 31)
    ku = xi ^ (s | jnp.int32(-2147483648))
    return lax.shift_right_logical(ku, 16)


def _suffix_scan2(ha, hb):
    def step(i, carry):
        runa, ba, runb, bb = carry
        d = NB - 1 - i
        cnta = jnp.sum(ha[pl.ds(d * 16, 16)])
        cntb = jnp.sum(hb[pl.ds(d * 16, 16)])
        nruna = runa + cnta
        nrunb = runb + cntb
        ba = jnp.where((runa < K) & (nruna >= K), d, ba)
        bb = jnp.where((runb < K) & (nrunb >= K), d, bb)
        return (nruna, ba, nrunb, bb)
    z = jnp.int32(0)
    _, ba, _, bb = lax.fori_loop(
        0, NB, step, (z, z, z, z), unroll=4)
    return ba, bb


def _body(x_hbm, mask_hbm, idx_hbm,
          row_a, row_b, cva, cvb, cia, cib,
          h1a, h1b, pva, pvb, oia, oib):
    cid = lax.axis_index("c")
    sid = lax.axis_index("s")
    wid = sid * 2 + cid
    lanes = lax.iota(jnp.int32, 16)
    ones = jnp.ones((16,), jnp.int32)
    zeros_i = jnp.zeros((16,), jnp.int32)
    zeros_f = jnp.zeros((16,), jnp.float32)
    ones_f = jnp.ones((16,), jnp.float32)
    ninf_v = jnp.full((16,), NEG_INF, jnp.float32)
    lane0 = lanes == 0
    z32 = jnp.int32(0)

    def pair_loop(pp, _):
        ra = wid * R_PER + pp * 2
        rb = ra + 1
        pltpu.sync_copy(x_hbm.at[pl.ds(ra * N, N)], row_a)
        pltpu.sync_copy(x_hbm.at[pl.ds(rb * N, N)], row_b)

        def zh(i, _):
            h1a[pl.ds(i * 16, 16)] = zeros_i
            h1b[pl.ds(i * 16, 16)] = zeros_i
            return 0
        lax.fori_loop(0, NB, zh, 0, unroll=4)

        def hist1(i, _):
            ka = _key16(row_a[pl.ds(i * 16, 16)])
            kb = _key16(row_b[pl.ds(i * 16, 16)])
            d1a = lax.shift_right_logical(ka, 6)
            d1b = lax.shift_right_logical(kb, 6)
            plsc.addupdate_scatter(h1a, [d1a * 16 + lanes], ones)
            plsc.addupdate_scatter(h1b, [d1b * 16 + lanes], ones)
            return 0
        lax.fori_loop(0, N // 16, hist1, 0, unroll=8)

        b1a, b1b = _suffix_scan2(h1a, h1b)
        tha = b1a * 64
        thb = b1b * 64

        def pf(i, _):
            cva[pl.ds(i * 16, 16)] = ninf_v
            cvb[pl.ds(i * 16, 16)] = ninf_v
            return 0
        lax.fori_loop(0, NVREG, pf, 0, unroll=4)

        H = N // 32
        seg_hi = jnp.full((16,), SEG - 16, jnp.int32)
        cap_hi = jnp.full((16,), CAP - 16, jnp.int32)
        def one_comp(row, cv, ci, off, i0, i, th, hi):
            v = row[pl.ds((i0 + i) * 16, 16)]
            m = _key16(v) >= th
            rank = plsc.cumsum(m.astype(jnp.int32)) - 1
            t = off + rank
            plsc.store_scatter(cv, [t], v, mask=m)
            plsc.store_scatter(ci, [t], (i0 + i) * 16 + lanes, mask=m)
            return jnp.minimum(off + plsc.all_reduce_population_count(m),
                               hi)
        def comp(i, offs):
            oa1, oa2, ob1, ob2 = offs
            oa1 = one_comp(row_a, cva, cia, oa1, 0, i, tha, seg_hi)
            oa2 = one_comp(row_a, cva, cia, oa2, H, i, tha, cap_hi)
            ob1 = one_comp(row_b, cvb, cib, ob1, 0, i, thb, seg_hi)
            ob2 = one_comp(row_b, cvb, cib, ob2, H, i, thb, cap_hi)
            return (oa1, oa2, ob1, ob2)
        zv = jnp.zeros((16,), jnp.int32)
        sv = jnp.full((16,), SEG, jnp.int32)
        lax.fori_loop(0, H, comp, (zv, sv, zv, sv), unroll=4)

        def pvi(i, _):
            mxa = jnp.max(cva[pl.ds(i * 16, 16)])
            mxb = jnp.max(cvb[pl.ds(i * 16, 16)])
            iv = jnp.broadcast_to(i, (16,))
            plsc.store_scatter(pva, [iv], jnp.broadcast_to(mxa, (16,)),
                               mask=lane0)
            plsc.store_scatter(pvb, [iv], jnp.broadcast_to(mxb, (16,)),
                               mask=lane0)
            return 0
        lax.fori_loop(0, NVREG, pvi, 0, unroll=4)

        def zm(i, _):
            row_a[pl.ds(i * 16, 16)] = zeros_f
            row_b[pl.ds(i * 16, 16)] = zeros_f
            return 0
        lax.fori_loop(0, N // 16, zm, 0, unroll=8)

        def ext1(pv, cv, ci, oi, k):
            ps = [pv[pl.ds(16 * t, 16)] for t in range(NVREG // 16)]
            mx = ps[0]
            for p in ps[1:]:
                mx = jnp.maximum(mx, p)
            best = jnp.max(mx)
            bestv = jnp.broadcast_to(best, (16,))
            qs = [jnp.where(p == bestv, lanes + 16 * t, BIG)
                  for t, p in enumerate(ps)]
            mn = qs[0]
            for q in qs[1:]:
                mn = jnp.minimum(mn, q)
            j = jnp.min(mn)
            vv = cv[pl.ds(j * 16, 16)]
            pos = jnp.min(jnp.where(vv == bestv, j * 16 + lanes, BIG))
            posv = jnp.broadcast_to(pos, (16,))
            idxv = plsc.load_gather(ci, [posv])
            plsc.store_scatter(oi, [jnp.broadcast_to(k, (16,))], idxv,
                               mask=lane0)
            plsc.store_scatter(cv, [posv], ninf_v, mask=lane0)
            vv2 = jnp.where(lanes == pos - j * 16, ninf_v, vv)
            plsc.store_scatter(pv, [jnp.broadcast_to(j, (16,))],
                               jnp.broadcast_to(jnp.max(vv2), (16,)),
                               mask=lane0)

        def ext(k, _):
            ext1(pva, cva, cia, oia, k)
            ext1(pvb, cvb, cib, oib, k)
            return 0
        lax.fori_loop(0, K, ext, 0)

        def msc(j, _):
            plsc.store_scatter(row_a, [oia[pl.ds(j * 16, 16)]], ones_f)
            plsc.store_scatter(row_b, [oib[pl.ds(j * 16, 16)]], ones_f)
            return 0
        lax.fori_loop(0, K // 16, msc, 0, unroll=4)

        pltpu.sync_copy(row_a, mask_hbm.at[pl.ds(ra * N, N)])
        pltpu.sync_copy(row_b, mask_hbm.at[pl.ds(rb * N, N)])
        pltpu.sync_copy(oia, idx_hbm.at[pl.ds(ra * K, K)])
        pltpu.sync_copy(oib, idx_hbm.at[pl.ds(rb * K, K)])
        return 0

    lax.fori_loop(0, R_PER // 2, pair_loop, 0)


@jax.jit
def kernel(logits):
    xf = logits.reshape(-1)
    mesh = plsc.VectorSubcoreMesh(core_axis_name="c", subcore_axis_name="s")
    run = pl.kernel(
        _body, mesh=mesh,
        compiler_params=pltpu.CompilerParams(needs_layout_passes=False),
        out_type=[jax.ShapeDtypeStruct((B * N,), jnp.float32),
                  jax.ShapeDtypeStruct((B * K,), jnp.int32)],
        scratch_types=[
            pltpu.VMEM((N,), jnp.float32),
            pltpu.VMEM((N,), jnp.float32),
            pltpu.VMEM((CAP,), jnp.float32),
            pltpu.VMEM((CAP,), jnp.float32),
            pltpu.VMEM((CAP,), jnp.int32),
            pltpu.VMEM((CAP,), jnp.int32),
            pltpu.VMEM((NB * 16,), jnp.int32),
            pltpu.VMEM((NB * 16,), jnp.int32),
            pltpu.VMEM((NVREG,), jnp.float32),
            pltpu.VMEM((NVREG,), jnp.float32),
            pltpu.VMEM((K,), jnp.int32),
            pltpu.VMEM((K,), jnp.int32),
        ])
    mask_f, idx_f = run(xf)
    return mask_f.reshape(B, N), idx_f.reshape(B, K)


# speedup vs baseline: 1.3580x; 1.3580x over previous
import jax
import jax.numpy as jnp
from jax import lax
from jax.experimental import pallas as pl
from jax.experimental.pallas import tpu as pltpu
from jax.experimental.pallas import tpu_sc as plsc

B = 128
N = 32768
K = 256
NW = 32
R_PER = B // NW
NB = 1024
SEG = 768
CAP = 2 * SEG
NVREG = CAP // 16
NEG_INF = float("-inf")
BIG = 2 ** 30


def _key16(v):
    xi = lax.bitcast_convert_type(v, jnp.int32)
    s = lax.shift_right_arithmetic(xi, 31)
    ku = xi ^ (s | jnp.int32(-2147483648))
    return lax.shift_right_logical(ku, 16)


def _suffix_scan2(ha, hb):
    def step(i, carry):
        runa, ba, runb, bb = carry
        d = NB - 1 - i
        cnta = jnp.sum(ha[pl.ds(d * 16, 16)])
        cntb = jnp.sum(hb[pl.ds(d * 16, 16)])
        nruna = runa + cnta
        nrunb = runb + cntb
        ba = jnp.where((runa < K) & (nruna >= K), d, ba)
        bb = jnp.where((runb < K) & (nrunb >= K), d, bb)
        return (nruna, ba, nrunb, bb)
    z = jnp.int32(0)
    _, ba, _, bb = lax.fori_loop(
        0, NB, step, (z, z, z, z), unroll=4)
    return ba, bb


def _body(x_hbm, mask_hbm, idx_hbm,
          row_a, row_b, cva, cvb, cia, cib,
          h1a, h1b, pva, pvb, oia, oib, sem1, sem2, sem3, sem4):
    cid = lax.axis_index("c")
    sid = lax.axis_index("s")
    wid = sid * 2 + cid
    lanes = lax.iota(jnp.int32, 16)
    ones = jnp.ones((16,), jnp.int32)
    zeros_i = jnp.zeros((16,), jnp.int32)
    zeros_f = jnp.zeros((16,), jnp.float32)
    ones_f = jnp.ones((16,), jnp.float32)
    ninf_v = jnp.full((16,), NEG_INF, jnp.float32)
    lane0 = lanes == 0
    z32 = jnp.int32(0)

    def pair_loop(pp, _):
        ra = wid * R_PER + pp * 2
        rb = ra + 1
        cpa = pltpu.async_copy(x_hbm.at[pl.ds(ra * N, N)], row_a, sem1)
        cpb = pltpu.async_copy(x_hbm.at[pl.ds(rb * N, N)], row_b, sem2)
        cpa.wait()
        cpb.wait()

        def zh(i, _):
            h1a[pl.ds(i * 16, 16)] = zeros_i
            h1b[pl.ds(i * 16, 16)] = zeros_i
            return 0
        lax.fori_loop(0, NB, zh, 0, unroll=4)

        def hist1(i, _):
            ka = _key16(row_a[pl.ds(i * 16, 16)])
            kb = _key16(row_b[pl.ds(i * 16, 16)])
            d1a = lax.shift_right_logical(ka, 6)
            d1b = lax.shift_right_logical(kb, 6)
            plsc.addupdate_scatter(h1a, [d1a * 16 + lanes], ones)
            plsc.addupdate_scatter(h1b, [d1b * 16 + lanes], ones)
            return 0
        lax.fori_loop(0, N // 16, hist1, 0, unroll=4)

        b1a, b1b = _suffix_scan2(h1a, h1b)
        tha = b1a * 64
        thb = b1b * 64

        def pf(i, _):
            cva[pl.ds(i * 16, 16)] = ninf_v
            cvb[pl.ds(i * 16, 16)] = ninf_v
            return 0
        lax.fori_loop(0, NVREG, pf, 0, unroll=4)

        H = N // 32
        def comp(i, offs):
            oa1, oa2, ob1, ob2 = offs
            va1 = row_a[pl.ds(i * 16, 16)]
            va2 = row_a[pl.ds((H + i) * 16, 16)]
            vb1 = row_b[pl.ds(i * 16, 16)]
            vb2 = row_b[pl.ds((H + i) * 16, 16)]
            ma1 = _key16(va1) >= tha
            ma2 = _key16(va2) >= tha
            mb1 = _key16(vb1) >= thb
            mb2 = _key16(vb2) >= thb
            iv1 = i * 16 + lanes
            iv2 = (H + i) * 16 + lanes
            plsc.store_compressed(cva.at[pl.ds(oa1, 16)], va1, mask=ma1)
            plsc.store_compressed(cia.at[pl.ds(oa1, 16)], iv1, mask=ma1)
            plsc.store_compressed(cva.at[pl.ds(oa2, 16)], va2, mask=ma2)
            plsc.store_compressed(cia.at[pl.ds(oa2, 16)], iv2, mask=ma2)
            plsc.store_compressed(cvb.at[pl.ds(ob1, 16)], vb1, mask=mb1)
            plsc.store_compressed(cib.at[pl.ds(ob1, 16)], iv1, mask=mb1)
            plsc.store_compressed(cvb.at[pl.ds(ob2, 16)], vb2, mask=mb2)
            plsc.store_compressed(cib.at[pl.ds(ob2, 16)], iv2, mask=mb2)
            pa1 = plsc.all_reduce_population_count(ma1)[0]
            pa2 = plsc.all_reduce_population_count(ma2)[0]
            pb1 = plsc.all_reduce_population_count(mb1)[0]
            pb2 = plsc.all_reduce_population_count(mb2)[0]
            return (jnp.minimum(oa1 + pa1, SEG - 16),
                    jnp.minimum(oa2 + pa2, CAP - 16),
                    jnp.minimum(ob1 + pb1, SEG - 16),
                    jnp.minimum(ob2 + pb2, CAP - 16))
        lax.fori_loop(0, H, comp,
                      (z32, jnp.int32(SEG), z32, jnp.int32(SEG)),
                      unroll=4)

        def pvi(i, _):
            mxa = jnp.max(cva[pl.ds(i * 16, 16)])
            mxb = jnp.max(cvb[pl.ds(i * 16, 16)])
            iv = jnp.broadcast_to(i, (16,))
            plsc.store_scatter(pva, [iv], jnp.broadcast_to(mxa, (16,)),
                               mask=lane0)
            plsc.store_scatter(pvb, [iv], jnp.broadcast_to(mxb, (16,)),
                               mask=lane0)
            return 0
        lax.fori_loop(0, NVREG, pvi, 0, unroll=4)

        def zm(i, _):
            row_a[pl.ds(i * 16, 16)] = zeros_f
            row_b[pl.ds(i * 16, 16)] = zeros_f
            return 0
        lax.fori_loop(0, N // 16, zm, 0, unroll=8)

        def ext1(pv, cv, ci, oi, k):
            ps = [pv[pl.ds(16 * t, 16)] for t in range(NVREG // 16)]
            mx = ps[0]
            for p in ps[1:]:
                mx = jnp.maximum(mx, p)
            best = jnp.max(mx)
            bestv = jnp.broadcast_to(best, (16,))
            qs = [jnp.where(p == bestv, lanes + 16 * t, BIG)
                  for t, p in enumerate(ps)]
            mn = qs[0]
            for q in qs[1:]:
                mn = jnp.minimum(mn, q)
            j = jnp.min(mn)
            vv = cv[pl.ds(j * 16, 16)]
            pos = jnp.min(jnp.where(vv == bestv, j * 16 + lanes, BIG))
            posv = jnp.broadcast_to(pos, (16,))
            idxv = plsc.load_gather(ci, [posv])
            plsc.store_scatter(oi, [jnp.broadcast_to(k, (16,))], idxv,
                               mask=lane0)
            plsc.store_scatter(cv, [posv], ninf_v, mask=lane0)
            vv2 = jnp.where(lanes == pos - j * 16, ninf_v, vv)
            plsc.store_scatter(pv, [jnp.broadcast_to(j, (16,))],
                               jnp.broadcast_to(jnp.max(vv2), (16,)),
                               mask=lane0)

        def ext(k, _):
            ext1(pva, cva, cia, oia, k)
            ext1(pvb, cvb, cib, oib, k)
            return 0
        lax.fori_loop(0, K, ext, 0)

        def msc(j, _):
            plsc.store_scatter(row_a, [oia[pl.ds(j * 16, 16)]], ones_f)
            plsc.store_scatter(row_b, [oib[pl.ds(j * 16, 16)]], ones_f)
            return 0
        lax.fori_loop(0, K // 16, msc, 0, unroll=4)

        o1 = pltpu.async_copy(row_a, mask_hbm.at[pl.ds(ra * N, N)], sem1)
        o2 = pltpu.async_copy(row_b, mask_hbm.at[pl.ds(rb * N, N)], sem2)
        o3 = pltpu.async_copy(oia, idx_hbm.at[pl.ds(ra * K, K)], sem3)
        o4 = pltpu.async_copy(oib, idx_hbm.at[pl.ds(rb * K, K)], sem4)
        o1.wait()
        o2.wait()
        o3.wait()
        o4.wait()
        return 0

    lax.fori_loop(0, R_PER // 2, pair_loop, 0)


@jax.jit
def kernel(logits):
    xf = logits.reshape(-1)
    mesh = plsc.VectorSubcoreMesh(core_axis_name="c", subcore_axis_name="s")
    run = pl.kernel(
        _body, mesh=mesh,
        compiler_params=pltpu.CompilerParams(needs_layout_passes=False),
        out_type=[jax.ShapeDtypeStruct((B * N,), jnp.float32),
                  jax.ShapeDtypeStruct((B * K,), jnp.int32)],
        scratch_types=[
            pltpu.VMEM((N,), jnp.float32),
            pltpu.VMEM((N,), jnp.float32),
            pltpu.VMEM((CAP,), jnp.float32),
            pltpu.VMEM((CAP,), jnp.float32),
            pltpu.VMEM((CAP,), jnp.int32),
            pltpu.VMEM((CAP,), jnp.int32),
            pltpu.VMEM((NB * 16,), jnp.int32),
            pltpu.VMEM((NB * 16,), jnp.int32),
            pltpu.VMEM((NVREG,), jnp.float32),
            pltpu.VMEM((NVREG,), jnp.float32),
            pltpu.VMEM((K,), jnp.int32),
            pltpu.VMEM((K,), jnp.int32),
            pltpu.SemaphoreType.DMA,
            pltpu.SemaphoreType.DMA,
            pltpu.SemaphoreType.DMA,
            pltpu.SemaphoreType.DMA,
        ])
    mask_f, idx_f = run(xf)
    return mask_f.reshape(B, N), idx_f.reshape(B, K)
